# baseline (speedup 1.0000x reference)
import functools

import jax
import jax.numpy as jnp
from jax import lax
from jax.experimental import pallas as pl
from jax.experimental.pallas import tpu as pltpu
from jax.experimental.pallas import tpu_sc as plsc

N = 10000
E = 160000
M = 256
C = 128
NEG_SLOPE = 0.2


_HALF_ROWS = N // 2
_HALF_WORDS = _HALF_ROWS * M
_DUMMY = _HALF_WORDS
_EPC = E // 16
_NGRP = _EPC // 16
_NROW = 80
_ZW = 16000


def _build_h_body(idx_n_hbm, idx_e_hbm, zeros_hbm, h_hbm, idx_n_v, idx_e_v,
                  fidx, ones_v, h_half, dsem, zsem):
    cid = lax.axis_index("c")
    sid = lax.axis_index("s")
    n0 = cid * _HALF_ROWS

    base_e = sid * _EPC
    cp_n = pltpu.async_copy(idx_n_hbm.at[pl.ds(base_e, _EPC)],
                            idx_n_v.at[pl.ds(0, _EPC)], zsem)
    cp_e = pltpu.async_copy(idx_e_hbm.at[pl.ds(base_e, _EPC)],
                            idx_e_v.at[pl.ds(0, _EPC)], zsem)

    for j in range(8):
        ones_v[pl.ds(j * 16, 16)] = jnp.ones((16,), jnp.float32)

    zbase = sid * (_HALF_WORDS // 16)
    zcp = pltpu.async_copy(zeros_hbm.at[pl.ds(zbase, _HALF_WORDS // 16)],
                           h_half.at[pl.ds(zbase, _HALF_WORDS // 16)], dsem)

    cp_n.wait()
    cp_e.wait()
    for j in range(15):
        idx_n_v[pl.ds(_EPC + j * 16, 16)] = jnp.full((16,), -1, jnp.int32)

    def _fx(r, _):
        for q in range(8):
            i = r * 8 + q
            n = idx_n_v[pl.ds(i * 16, 16)]
            m = idx_e_v[pl.ds(i * 16, 16)]
            in_half = (n >= n0) & (n < n0 + _HALF_ROWS)
            f = jnp.where(in_half, (n - n0) * M + m,
                          jnp.full((16,), _DUMMY, jnp.int32))
            fidx[r, pl.ds(q * 16, 16)] = f
        return 0
    lax.fori_loop(0, _NROW, _fx, 0)

    zcp.wait()

    plsc.subcore_barrier()

    def _sc(rnd, _):
        for q in range(16):
            j = rnd * 16 + q
            pltpu.async_copy(ones_v, h_half.at[fidx.at[j]], dsem, add=True)
        for q in range(16):
            j = rnd * 16 + q
            pltpu.make_async_copy(ones_v, h_half.at[fidx.at[j]], dsem).wait()
        return 0
    lax.fori_loop(0, _NROW // 16, _sc, 0)

    plsc.subcore_barrier()

    out_base = cid * _HALF_WORDS + sid * (_HALF_WORDS // 16)
    pltpu.sync_copy(h_half.at[pl.ds(sid * (_HALF_WORDS // 16),
                                    _HALF_WORDS // 16)],
                    h_hbm.at[pl.ds(out_base, _HALF_WORDS // 16)])


def _build_h(idx_n, idx_e):
    mesh = plsc.VectorSubcoreMesh(core_axis_name="c", subcore_axis_name="s")
    zeros_hbm = jnp.zeros((_HALF_WORDS,), jnp.float32)
    return pl.kernel(
        _build_h_body,
        out_type=jax.ShapeDtypeStruct((N * M,), jnp.float32),
        mesh=mesh,
        scratch_types=[
            pltpu.VMEM((_EPC + 240,), jnp.int32),
            pltpu.VMEM((_EPC + 240,), jnp.int32),
            pltpu.VMEM((_NROW, 128), jnp.int32),
            pltpu.VMEM((128,), jnp.float32),
            pltpu.VMEM_SHARED((_HALF_WORDS + 16,), jnp.float32),
            pltpu.SemaphoreType.DMA,
            pltpu.SemaphoreType.DMA,
        ],
    )(idx_n, idx_e, zeros_hbm)


_NB = 2000
_GN = N // _NB
_KB = 32
_GK = M // _KB


def _tc1_body(x_ref, w_ref, h_ref, xw_ref, d_ref, es_ref, bdeg_ref, du_ref):
    i = pl.program_id(0)
    xw = jnp.dot(x_ref[...], w_ref[...], preferred_element_type=jnp.float32)
    xw_ref[...] = xw
    h = h_ref[...]
    dcol = jnp.sum(h, axis=1, keepdims=True)
    d_ref[...] = dcol
    es_p = lax.dot_general(h, xw, (((0,), (0,)), ((), ())),
                           preferred_element_type=jnp.float32,
                           precision=lax.Precision.HIGHEST)
    bdeg_p = jnp.sum(h, axis=0, keepdims=True)
    u = jnp.sum(xw, axis=1, keepdims=True)
    du_p = jnp.reshape(jnp.sum(dcol * u), (1, 1))

    @pl.when(i == 0)
    def _init():
        es_ref[...] = es_p
        bdeg_ref[...] = bdeg_p
        du_ref[...] = du_p

    @pl.when(i > 0)
    def _acc():
        es_ref[...] += es_p
        bdeg_ref[...] += bdeg_p
        du_ref[...] += du_p


def _tc1(x2d, weight, h2d):
    return pl.pallas_call(
        _tc1_body,
        grid=(_GN,),
        in_specs=[
            pl.BlockSpec((_NB, C), lambda i: (i, 0)),
            pl.BlockSpec((C, C), lambda i: (0, 0)),
            pl.BlockSpec((_NB, M), lambda i: (i, 0)),
        ],
        out_specs=[
            pl.BlockSpec((_NB, C), lambda i: (i, 0)),
            pl.BlockSpec((_NB, 1), lambda i: (i, 0)),
            pl.BlockSpec((M, C), lambda i: (0, 0)),
            pl.BlockSpec((1, M), lambda i: (0, 0)),
            pl.BlockSpec((1, 1), lambda i: (0, 0)),
        ],
        out_shape=[
            jax.ShapeDtypeStruct((N, C), jnp.float32),
            jax.ShapeDtypeStruct((N, 1), jnp.float32),
            jax.ShapeDtypeStruct((M, C), jnp.float32),
            jax.ShapeDtypeStruct((1, M), jnp.float32),
            jax.ShapeDtypeStruct((1, 1), jnp.float32),
        ],
    )(x2d, weight, h2d)


def _tc2_body(esk_ref, es_ref, bdeg_ref, loss_ref, bv_ref):
    k = pl.program_id(0)
    esk = esk_ref[...]
    es = es_ref[...]
    diff = esk[:, None, :] - es[None, :, :]
    ssq = jnp.sum(diff * diff, axis=-1)
    dist = jnp.sqrt(ssq + 1e-12)
    inner = jnp.sum(esk[:, None, :] * es[None, :, :], axis=-1)
    nq_col = jnp.sqrt(jnp.sum(esk * esk, axis=1, keepdims=True))
    nq_row = jnp.sqrt(jnp.sum(es * es, axis=1))[None, :]
    alpha = inner / (nq_col * nq_row)
    item = alpha * dist + (1.0 - alpha) * jnp.maximum(4.2 - dist, 0.0)
    loss_p = jnp.reshape(jnp.sum(jnp.abs(item)), (1, 1))

    @pl.when(k == 0)
    def _init():
        loss_ref[...] = loss_p
        v = jnp.sum(es, axis=1)[None, :]
        bv_ref[...] = jnp.reshape(jnp.sum(bdeg_ref[...] * v), (1, 1))

    @pl.when(k > 0)
    def _acc():
        loss_ref[...] += loss_p


def _tc2(es, bdeg):
    return pl.pallas_call(
        _tc2_body,
        grid=(_GK,),
        in_specs=[
            pl.BlockSpec((_KB, C), lambda k: (k, 0)),
            pl.BlockSpec((M, C), lambda k: (0, 0)),
            pl.BlockSpec((1, M), lambda k: (0, 0)),
        ],
        out_specs=[
            pl.BlockSpec((1, 1), lambda k: (0, 0)),
            pl.BlockSpec((1, 1), lambda k: (0, 0)),
        ],
        out_shape=[
            jax.ShapeDtypeStruct((1, 1), jnp.float32),
            jax.ShapeDtypeStruct((1, 1), jnp.float32),
        ],
    )(es, es, bdeg)


def _tc3_body(h_ref, xw_ref, es_ref, a1_ref, a2_ref, a_out_ref, ax_ref):
    i = pl.program_id(0)
    h = h_ref[...]
    xw = xw_ref[...]
    s_col = jnp.sum(xw * a1_ref[...], axis=1, keepdims=True)
    t_row = jnp.sum(es_ref[...] * a2_ref[...], axis=1)[None, :]
    r = s_col + t_row
    r = jnp.where(r >= 0, r, NEG_SLOPE * r)
    mask = h > 0
    rm = jnp.where(mask, r, -jnp.inf)
    nmax = jnp.max(rm, axis=1, keepdims=True)
    nmax = jnp.where(nmax > -3e38, nmax, 0.0)
    ex = jnp.where(mask, jnp.exp(r - nmax), 0.0)
    nsum = jnp.sum(h * ex, axis=1, keepdims=True)
    a = h * ex / (nsum + 1e-16)
    a_out_ref[...] = a
    ax_p = lax.dot_general(a, xw, (((0,), (0,)), ((), ())),
                           preferred_element_type=jnp.float32,
                           precision=lax.Precision.HIGHEST)

    @pl.when(i == 0)
    def _init():
        ax_ref[...] = ax_p

    @pl.when(i > 0)
    def _acc():
        ax_ref[...] += ax_p


def _tc3(h2d, xw, es, a1, a2):
    return pl.pallas_call(
        _tc3_body,
        grid=(_GN,),
        in_specs=[
            pl.BlockSpec((_NB, M), lambda i: (i, 0)),
            pl.BlockSpec((_NB, C), lambda i: (i, 0)),
            pl.BlockSpec((M, C), lambda i: (0, 0)),
            pl.BlockSpec((1, C), lambda i: (0, 0)),
            pl.BlockSpec((1, C), lambda i: (0, 0)),
        ],
        out_specs=[
            pl.BlockSpec((_NB, M), lambda i: (i, 0)),
            pl.BlockSpec((M, C), lambda i: (0, 0)),
        ],
        out_shape=[
            jax.ShapeDtypeStruct((N, M), jnp.float32),
            jax.ShapeDtypeStruct((M, C), jnp.float32),
        ],
    )(h2d, xw, es, a1, a2)


def _tc4_body(a_ref, ax_ref, bdeg_ref, d_ref, out_ref):
    bdeg = bdeg_ref[...]
    bn_row = jnp.where(bdeg > 0, 1.0 / bdeg, 0.0)
    a_scaled = a_ref[...] * bn_row
    o = jnp.dot(a_scaled, ax_ref[...], preferred_element_type=jnp.float32,
                precision=lax.Precision.HIGHEST)
    out_ref[...] = d_ref[...] * o


def _tc4(a_mat, ax, bdeg, d_col):
    return pl.pallas_call(
        _tc4_body,
        grid=(_GN,),
        in_specs=[
            pl.BlockSpec((_NB, M), lambda i: (i, 0)),
            pl.BlockSpec((M, C), lambda i: (0, 0)),
            pl.BlockSpec((1, M), lambda i: (0, 0)),
            pl.BlockSpec((_NB, 1), lambda i: (i, 0)),
        ],
        out_specs=pl.BlockSpec((_NB, C), lambda i: (i, 0)),
        out_shape=jax.ShapeDtypeStruct((N, C), jnp.float32),
    )(a_mat, ax, bdeg, d_col)


def kernel(x, hyperedge_index, weight, att):
    x2d = x[0]
    idx_n = hyperedge_index[0]
    idx_e = hyperedge_index[1]
    a1 = att[0, :, :C].reshape(1, C)
    a2 = att[0, :, C:].reshape(1, C)

    h_flat = _build_h(idx_n, idx_e)
    h2d = h_flat.reshape(N, M)

    xw, d_col, es, bdeg, du = _tc1(x2d, weight, h2d)
    loss, bv = _tc2(es, bdeg)
    a_mat, ax = _tc3(h2d, xw, es, a1, a2)
    out2 = _tc4(a_mat, ax, bdeg, d_col)

    out = out2[None, :, :]
    constrain = (jnp.abs((du[0, 0] - bv[0, 0]) / float(E * C))
                 + loss[0, 0] / float((M + 1) ** 2))
    return out, constrain


# speedup vs baseline: 1.5928x; 1.5928x over previous
import functools

import jax
import jax.numpy as jnp
from jax import lax
from jax.experimental import pallas as pl
from jax.experimental.pallas import tpu as pltpu
from jax.experimental.pallas import tpu_sc as plsc

N = 10000
E = 160000
M = 256
C = 128
NEG_SLOPE = 0.2


_HALF_ROWS = N // 2
_HALF_WORDS = _HALF_ROWS * M
_DUMMY = _HALF_WORDS
_EPC = E // 16
_NGRP = _EPC // 16
_NROW = 80
_ZW = 16000


def _build_h_body(idx_n_hbm, idx_e_hbm, zeros_hbm, h_hbm, idx_n_v, idx_e_v,
                  fidx, ones_v, h_half, dsem, zsem):
    cid = lax.axis_index("c")
    sid = lax.axis_index("s")
    n0 = cid * _HALF_ROWS

    base_e = sid * _EPC
    cp_n = pltpu.async_copy(idx_n_hbm.at[pl.ds(base_e, _EPC)],
                            idx_n_v.at[pl.ds(0, _EPC)], zsem)
    cp_e = pltpu.async_copy(idx_e_hbm.at[pl.ds(base_e, _EPC)],
                            idx_e_v.at[pl.ds(0, _EPC)], zsem)

    for j in range(8):
        ones_v[pl.ds(j * 16, 16)] = jnp.ones((16,), jnp.float32)

    zbase = sid * (_HALF_WORDS // 16)
    zcp = pltpu.async_copy(zeros_hbm.at[pl.ds(zbase, _HALF_WORDS // 16)],
                           h_half.at[pl.ds(zbase, _HALF_WORDS // 16)], dsem)

    cp_n.wait()
    cp_e.wait()
    for j in range(15):
        idx_n_v[pl.ds(_EPC + j * 16, 16)] = jnp.full((16,), -1, jnp.int32)

    lane = lax.iota(jnp.int32, 16)

    def _fx(r, _):
        for q in range(8):
            i = r * 8 + q
            n = idx_n_v[pl.ds(i * 16, 16)]
            m = idx_e_v[pl.ds(i * 16, 16)]
            in_half = (n >= n0) & (n < n0 + _HALF_ROWS)
            dummy = _DUMMY + ((i * 16 + lane) & 2047)
            f = jnp.where(in_half, (n - n0) * M + m, dummy)
            fidx[r, pl.ds(q * 16, 16)] = f
        return 0
    lax.fori_loop(0, _NROW, _fx, 0)

    zcp.wait()

    plsc.subcore_barrier()

    def _fire(j, _):
        pltpu.async_copy(ones_v, h_half.at[fidx.at[j]], dsem, add=True)
        return 0
    lax.fori_loop(0, _NROW, _fire, 0)

    def _drain(j, _):
        pltpu.make_async_copy(ones_v, h_half.at[fidx.at[j]], dsem).wait()
        return 0
    lax.fori_loop(0, _NROW, _drain, 0)

    plsc.subcore_barrier()

    out_base = cid * _HALF_WORDS + sid * (_HALF_WORDS // 16)
    pltpu.sync_copy(h_half.at[pl.ds(sid * (_HALF_WORDS // 16),
                                    _HALF_WORDS // 16)],
                    h_hbm.at[pl.ds(out_base, _HALF_WORDS // 16)])


def _build_h(idx_n, idx_e):
    mesh = plsc.VectorSubcoreMesh(core_axis_name="c", subcore_axis_name="s")
    zeros_hbm = jnp.zeros((_HALF_WORDS,), jnp.float32)
    return pl.kernel(
        _build_h_body,
        out_type=jax.ShapeDtypeStruct((N * M,), jnp.float32),
        mesh=mesh,
        scratch_types=[
            pltpu.VMEM((_EPC + 240,), jnp.int32),
            pltpu.VMEM((_EPC + 240,), jnp.int32),
            pltpu.VMEM((_NROW, 128), jnp.int32),
            pltpu.VMEM((128,), jnp.float32),
            pltpu.VMEM_SHARED((_HALF_WORDS + 2048,), jnp.float32),
            pltpu.SemaphoreType.DMA,
            pltpu.SemaphoreType.DMA,
        ],
    )(idx_n, idx_e, zeros_hbm)


_NB = 2000
_GN = N // _NB
_KB = 32
_GK = M // _KB


def _tc1_body(x_ref, w_ref, h_ref, xw_ref, d_ref, es_ref, bdeg_ref, du_ref):
    i = pl.program_id(0)
    xw = jnp.dot(x_ref[...], w_ref[...], preferred_element_type=jnp.float32)
    xw_ref[...] = xw
    h = h_ref[...]
    dcol = jnp.sum(h, axis=1, keepdims=True)
    d_ref[...] = dcol
    es_p = lax.dot_general(h, xw, (((0,), (0,)), ((), ())),
                           preferred_element_type=jnp.float32,
                           precision=lax.Precision.HIGHEST)
    bdeg_p = jnp.sum(h, axis=0, keepdims=True)
    u = jnp.sum(xw, axis=1, keepdims=True)
    du_p = jnp.reshape(jnp.sum(dcol * u), (1, 1))

    @pl.when(i == 0)
    def _init():
        es_ref[...] = es_p
        bdeg_ref[...] = bdeg_p
        du_ref[...] = du_p

    @pl.when(i > 0)
    def _acc():
        es_ref[...] += es_p
        bdeg_ref[...] += bdeg_p
        du_ref[...] += du_p


def _tc1(x2d, weight, h2d):
    return pl.pallas_call(
        _tc1_body,
        grid=(_GN,),
        in_specs=[
            pl.BlockSpec((_NB, C), lambda i: (i, 0)),
            pl.BlockSpec((C, C), lambda i: (0, 0)),
            pl.BlockSpec((_NB, M), lambda i: (i, 0)),
        ],
        out_specs=[
            pl.BlockSpec((_NB, C), lambda i: (i, 0)),
            pl.BlockSpec((_NB, 1), lambda i: (i, 0)),
            pl.BlockSpec((M, C), lambda i: (0, 0)),
            pl.BlockSpec((1, M), lambda i: (0, 0)),
            pl.BlockSpec((1, 1), lambda i: (0, 0)),
        ],
        out_shape=[
            jax.ShapeDtypeStruct((N, C), jnp.float32),
            jax.ShapeDtypeStruct((N, 1), jnp.float32),
            jax.ShapeDtypeStruct((M, C), jnp.float32),
            jax.ShapeDtypeStruct((1, M), jnp.float32),
            jax.ShapeDtypeStruct((1, 1), jnp.float32),
        ],
    )(x2d, weight, h2d)


def _tc2_body(esk_ref, es_ref, bdeg_ref, loss_ref, bv_ref):
    k = pl.program_id(0)
    esk = esk_ref[...]
    es = es_ref[...]
    diff = esk[:, None, :] - es[None, :, :]
    ssq = jnp.sum(diff * diff, axis=-1)
    dist = jnp.sqrt(ssq + 1e-12)
    inner = jnp.sum(esk[:, None, :] * es[None, :, :], axis=-1)
    nq_col = jnp.sqrt(jnp.sum(esk * esk, axis=1, keepdims=True))
    nq_row = jnp.sqrt(jnp.sum(es * es, axis=1))[None, :]
    alpha = inner / (nq_col * nq_row)
    item = alpha * dist + (1.0 - alpha) * jnp.maximum(4.2 - dist, 0.0)
    loss_p = jnp.reshape(jnp.sum(jnp.abs(item)), (1, 1))

    @pl.when(k == 0)
    def _init():
        loss_ref[...] = loss_p
        v = jnp.sum(es, axis=1)[None, :]
        bv_ref[...] = jnp.reshape(jnp.sum(bdeg_ref[...] * v), (1, 1))

    @pl.when(k > 0)
    def _acc():
        loss_ref[...] += loss_p


def _tc2(es, bdeg):
    return pl.pallas_call(
        _tc2_body,
        grid=(_GK,),
        in_specs=[
            pl.BlockSpec((_KB, C), lambda k: (k, 0)),
            pl.BlockSpec((M, C), lambda k: (0, 0)),
            pl.BlockSpec((1, M), lambda k: (0, 0)),
        ],
        out_specs=[
            pl.BlockSpec((1, 1), lambda k: (0, 0)),
            pl.BlockSpec((1, 1), lambda k: (0, 0)),
        ],
        out_shape=[
            jax.ShapeDtypeStruct((1, 1), jnp.float32),
            jax.ShapeDtypeStruct((1, 1), jnp.float32),
        ],
    )(es, es, bdeg)


def _tc3_body(h_ref, xw_ref, es_ref, a1_ref, a2_ref, a_out_ref, ax_ref):
    i = pl.program_id(0)
    h = h_ref[...]
    xw = xw_ref[...]
    s_col = jnp.sum(xw * a1_ref[...], axis=1, keepdims=True)
    t_row = jnp.sum(es_ref[...] * a2_ref[...], axis=1)[None, :]
    r = s_col + t_row
    r = jnp.where(r >= 0, r, NEG_SLOPE * r)
    mask = h > 0
    rm = jnp.where(mask, r, -jnp.inf)
    nmax = jnp.max(rm, axis=1, keepdims=True)
    nmax = jnp.where(nmax > -3e38, nmax, 0.0)
    ex = jnp.where(mask, jnp.exp(r - nmax), 0.0)
    nsum = jnp.sum(h * ex, axis=1, keepdims=True)
    a = h * ex / (nsum + 1e-16)
    a_out_ref[...] = a
    ax_p = lax.dot_general(a, xw, (((0,), (0,)), ((), ())),
                           preferred_element_type=jnp.float32,
                           precision=lax.Precision.HIGHEST)

    @pl.when(i == 0)
    def _init():
        ax_ref[...] = ax_p

    @pl.when(i > 0)
    def _acc():
        ax_ref[...] += ax_p


def _tc3(h2d, xw, es, a1, a2):
    return pl.pallas_call(
        _tc3_body,
        grid=(_GN,),
        in_specs=[
            pl.BlockSpec((_NB, M), lambda i: (i, 0)),
            pl.BlockSpec((_NB, C), lambda i: (i, 0)),
            pl.BlockSpec((M, C), lambda i: (0, 0)),
            pl.BlockSpec((1, C), lambda i: (0, 0)),
            pl.BlockSpec((1, C), lambda i: (0, 0)),
        ],
        out_specs=[
            pl.BlockSpec((_NB, M), lambda i: (i, 0)),
            pl.BlockSpec((M, C), lambda i: (0, 0)),
        ],
        out_shape=[
            jax.ShapeDtypeStruct((N, M), jnp.float32),
            jax.ShapeDtypeStruct((M, C), jnp.float32),
        ],
    )(h2d, xw, es, a1, a2)


def _tc4_body(a_ref, ax_ref, bdeg_ref, d_ref, out_ref):
    bdeg = bdeg_ref[...]
    bn_row = jnp.where(bdeg > 0, 1.0 / bdeg, 0.0)
    a_scaled = a_ref[...] * bn_row
    o = jnp.dot(a_scaled, ax_ref[...], preferred_element_type=jnp.float32,
                precision=lax.Precision.HIGHEST)
    out_ref[...] = d_ref[...] * o


def _tc4(a_mat, ax, bdeg, d_col):
    return pl.pallas_call(
        _tc4_body,
        grid=(_GN,),
        in_specs=[
            pl.BlockSpec((_NB, M), lambda i: (i, 0)),
            pl.BlockSpec((M, C), lambda i: (0, 0)),
            pl.BlockSpec((1, M), lambda i: (0, 0)),
            pl.BlockSpec((_NB, 1), lambda i: (i, 0)),
        ],
        out_specs=pl.BlockSpec((_NB, C), lambda i: (i, 0)),
        out_shape=jax.ShapeDtypeStruct((N, C), jnp.float32),
    )(a_mat, ax, bdeg, d_col)


def kernel(x, hyperedge_index, weight, att):
    x2d = x[0]
    idx_n = hyperedge_index[0]
    idx_e = hyperedge_index[1]
    a1 = att[0, :, :C].reshape(1, C)
    a2 = att[0, :, C:].reshape(1, C)

    h_flat = _build_h(idx_n, idx_e)
    h2d = h_flat.reshape(N, M)

    xw, d_col, es, bdeg, du = _tc1(x2d, weight, h2d)
    loss, bv = _tc2(es, bdeg)
    a_mat, ax = _tc3(h2d, xw, es, a1, a2)
    out2 = _tc4(a_mat, ax, bdeg, d_col)

    out = out2[None, :, :]
    constrain = (jnp.abs((du[0, 0] - bv[0, 0]) / float(E * C))
                 + loss[0, 0] / float((M + 1) ** 2))
    return out, constrain


# speedup vs baseline: 2.0921x; 1.3134x over previous
import functools

import jax
import jax.numpy as jnp
from jax import lax
from jax.experimental import pallas as pl
from jax.experimental.pallas import tpu as pltpu
from jax.experimental.pallas import tpu_sc as plsc

N = 10000
E = 160000
M = 256
C = 128
NEG_SLOPE = 0.2


_HALF_ROWS = N // 2
_HALF_WORDS = _HALF_ROWS * M
_DUMMY = _HALF_WORDS
_EPC = E // 16
_NGRP = _EPC // 16
_NROW = 80
_ZW = 16000


def _build_h_body(idx_n_hbm, idx_e_hbm, zeros_hbm, h_hbm, idx_n_v, idx_e_v,
                  fidx, ones_v, h_half, dsem, zsem):
    cid = lax.axis_index("c")
    sid = lax.axis_index("s")
    n0 = cid * _HALF_ROWS

    base_e = sid * _EPC
    cp_n = pltpu.async_copy(idx_n_hbm.at[pl.ds(base_e, _EPC)],
                            idx_n_v.at[pl.ds(0, _EPC)], zsem)
    cp_e = pltpu.async_copy(idx_e_hbm.at[pl.ds(base_e, _EPC)],
                            idx_e_v.at[pl.ds(0, _EPC)], zsem)

    for j in range(8):
        ones_v[pl.ds(j * 16, 16)] = jnp.ones((16,), jnp.float32)

    zbase = sid * (_HALF_WORDS // 16)
    zcp = pltpu.async_copy(zeros_hbm.at[pl.ds(zbase, _HALF_WORDS // 16)],
                           h_half.at[pl.ds(zbase, _HALF_WORDS // 16)], dsem)

    cp_n.wait()
    cp_e.wait()
    for j in range(15):
        idx_n_v[pl.ds(_EPC + j * 16, 16)] = jnp.full((16,), -1, jnp.int32)

    lane = lax.iota(jnp.int32, 16)

    def _fx(r, _):
        for q in range(8):
            i = r * 8 + q
            n = idx_n_v[pl.ds(i * 16, 16)]
            m = idx_e_v[pl.ds(i * 16, 16)]
            in_half = (n >= n0) & (n < n0 + _HALF_ROWS)
            dummy = _DUMMY + ((i * 16 + lane) & 2047)
            f = jnp.where(in_half, (n - n0) * M + m, dummy)
            fidx[r, pl.ds(q * 16, 16)] = f
        return 0
    lax.fori_loop(0, _NROW, _fx, 0)

    zcp.wait()

    plsc.subcore_barrier()

    def _fire(j, _):
        pltpu.async_copy(ones_v, h_half.at[fidx.at[j]], dsem, add=True)
        return 0
    lax.fori_loop(0, _NROW, _fire, 0)

    def _drain(j, _):
        pltpu.make_async_copy(ones_v, h_half.at[fidx.at[j]], dsem).wait()
        return 0
    lax.fori_loop(0, _NROW, _drain, 0)

    plsc.subcore_barrier()

    out_base = cid * _HALF_WORDS + sid * (_HALF_WORDS // 16)
    pltpu.sync_copy(h_half.at[pl.ds(sid * (_HALF_WORDS // 16),
                                    _HALF_WORDS // 16)],
                    h_hbm.at[pl.ds(out_base, _HALF_WORDS // 16)])


def _build_h(idx_n, idx_e):
    mesh = plsc.VectorSubcoreMesh(core_axis_name="c", subcore_axis_name="s")
    zeros_hbm = jnp.zeros((_HALF_WORDS,), jnp.float32)
    return pl.kernel(
        _build_h_body,
        out_type=jax.ShapeDtypeStruct((N * M,), jnp.float32),
        mesh=mesh,
        scratch_types=[
            pltpu.VMEM((_EPC + 240,), jnp.int32),
            pltpu.VMEM((_EPC + 240,), jnp.int32),
            pltpu.VMEM((_NROW, 128), jnp.int32),
            pltpu.VMEM((128,), jnp.float32),
            pltpu.VMEM_SHARED((_HALF_WORDS + 2048,), jnp.float32),
            pltpu.SemaphoreType.DMA,
            pltpu.SemaphoreType.DMA,
        ],
    )(idx_n, idx_e, zeros_hbm)


_NB = 2000
_GN = N // _NB
_KB = 32
_GK = M // _KB


def _tc1_body(x_ref, w_ref, h_ref, xw_ref, d_ref, es_ref, bdeg_ref, du_ref):
    i = pl.program_id(0)
    xw = jnp.dot(x_ref[...], w_ref[...], preferred_element_type=jnp.float32)
    xw_ref[...] = xw
    h = h_ref[...]
    dcol = jnp.sum(h, axis=1, keepdims=True)
    d_ref[...] = dcol
    es_p = lax.dot_general(h, xw, (((0,), (0,)), ((), ())),
                           preferred_element_type=jnp.float32,
                           precision=lax.Precision.HIGHEST)
    bdeg_p = jnp.sum(h, axis=0, keepdims=True)
    u = jnp.sum(xw, axis=1, keepdims=True)
    du_p = jnp.reshape(jnp.sum(dcol * u), (1, 1))

    @pl.when(i == 0)
    def _init():
        es_ref[...] = es_p
        bdeg_ref[...] = bdeg_p
        du_ref[...] = du_p

    @pl.when(i > 0)
    def _acc():
        es_ref[...] += es_p
        bdeg_ref[...] += bdeg_p
        du_ref[...] += du_p


def _tc1(x2d, weight, h2d):
    return pl.pallas_call(
        _tc1_body,
        grid=(_GN,),
        in_specs=[
            pl.BlockSpec((_NB, C), lambda i: (i, 0)),
            pl.BlockSpec((C, C), lambda i: (0, 0)),
            pl.BlockSpec((_NB, M), lambda i: (i, 0)),
        ],
        out_specs=[
            pl.BlockSpec((_NB, C), lambda i: (i, 0)),
            pl.BlockSpec((_NB, 1), lambda i: (i, 0)),
            pl.BlockSpec((M, C), lambda i: (0, 0)),
            pl.BlockSpec((1, M), lambda i: (0, 0)),
            pl.BlockSpec((1, 1), lambda i: (0, 0)),
        ],
        out_shape=[
            jax.ShapeDtypeStruct((N, C), jnp.float32),
            jax.ShapeDtypeStruct((N, 1), jnp.float32),
            jax.ShapeDtypeStruct((M, C), jnp.float32),
            jax.ShapeDtypeStruct((1, M), jnp.float32),
            jax.ShapeDtypeStruct((1, 1), jnp.float32),
        ],
    )(x2d, weight, h2d)


def _tc2_body(esk_ref, es_ref, bdeg_ref, loss_ref, bv_ref):
    k = pl.program_id(0)
    esk = esk_ref[...]
    es = es_ref[...]
    inner = lax.dot_general(esk, es, (((1,), (1,)), ((), ())),
                            preferred_element_type=jnp.float32,
                            precision=lax.Precision.HIGHEST)
    sq_col = jnp.sum(esk * esk, axis=1, keepdims=True)
    sq_row = jnp.sum(es * es, axis=1)[None, :]
    ssq = jnp.maximum(sq_col + sq_row - 2.0 * inner, 0.0)
    dist = jnp.sqrt(ssq + 1e-12)
    nq_col = jnp.sqrt(sq_col)
    nq_row = jnp.sqrt(sq_row)
    alpha = inner / (nq_col * nq_row)
    item = alpha * dist + (1.0 - alpha) * jnp.maximum(4.2 - dist, 0.0)
    loss_p = jnp.reshape(jnp.sum(jnp.abs(item)), (1, 1))

    @pl.when(k == 0)
    def _init():
        loss_ref[...] = loss_p
        v = jnp.sum(es, axis=1)[None, :]
        bv_ref[...] = jnp.reshape(jnp.sum(bdeg_ref[...] * v), (1, 1))

    @pl.when(k > 0)
    def _acc():
        loss_ref[...] += loss_p


def _tc2(es, bdeg):
    return pl.pallas_call(
        _tc2_body,
        grid=(_GK,),
        in_specs=[
            pl.BlockSpec((_KB, C), lambda k: (k, 0)),
            pl.BlockSpec((M, C), lambda k: (0, 0)),
            pl.BlockSpec((1, M), lambda k: (0, 0)),
        ],
        out_specs=[
            pl.BlockSpec((1, 1), lambda k: (0, 0)),
            pl.BlockSpec((1, 1), lambda k: (0, 0)),
        ],
        out_shape=[
            jax.ShapeDtypeStruct((1, 1), jnp.float32),
            jax.ShapeDtypeStruct((1, 1), jnp.float32),
        ],
    )(es, es, bdeg)


def _tc3_body(h_ref, xw_ref, es_ref, a1_ref, a2_ref, a_out_ref, ax_ref):
    i = pl.program_id(0)
    h = h_ref[...]
    xw = xw_ref[...]
    s_col = jnp.sum(xw * a1_ref[...], axis=1, keepdims=True)
    t_row = jnp.sum(es_ref[...] * a2_ref[...], axis=1)[None, :]
    r = s_col + t_row
    r = jnp.where(r >= 0, r, NEG_SLOPE * r)
    mask = h > 0
    rm = jnp.where(mask, r, -jnp.inf)
    nmax = jnp.max(rm, axis=1, keepdims=True)
    nmax = jnp.where(nmax > -3e38, nmax, 0.0)
    ex = jnp.where(mask, jnp.exp(r - nmax), 0.0)
    nsum = jnp.sum(h * ex, axis=1, keepdims=True)
    a = h * ex / (nsum + 1e-16)
    a_out_ref[...] = a
    ax_p = lax.dot_general(a, xw, (((0,), (0,)), ((), ())),
                           preferred_element_type=jnp.float32,
                           precision=lax.Precision.HIGHEST)

    @pl.when(i == 0)
    def _init():
        ax_ref[...] = ax_p

    @pl.when(i > 0)
    def _acc():
        ax_ref[...] += ax_p


def _tc3(h2d, xw, es, a1, a2):
    return pl.pallas_call(
        _tc3_body,
        grid=(_GN,),
        in_specs=[
            pl.BlockSpec((_NB, M), lambda i: (i, 0)),
            pl.BlockSpec((_NB, C), lambda i: (i, 0)),
            pl.BlockSpec((M, C), lambda i: (0, 0)),
            pl.BlockSpec((1, C), lambda i: (0, 0)),
            pl.BlockSpec((1, C), lambda i: (0, 0)),
        ],
        out_specs=[
            pl.BlockSpec((_NB, M), lambda i: (i, 0)),
            pl.BlockSpec((M, C), lambda i: (0, 0)),
        ],
        out_shape=[
            jax.ShapeDtypeStruct((N, M), jnp.float32),
            jax.ShapeDtypeStruct((M, C), jnp.float32),
        ],
    )(h2d, xw, es, a1, a2)


def _tc4_body(a_ref, ax_ref, bdeg_ref, d_ref, out_ref):
    bdeg = bdeg_ref[...]
    bn_row = jnp.where(bdeg > 0, 1.0 / bdeg, 0.0)
    a_scaled = a_ref[...] * bn_row
    o = jnp.dot(a_scaled, ax_ref[...], preferred_element_type=jnp.float32,
                precision=lax.Precision.HIGHEST)
    out_ref[...] = d_ref[...] * o


def _tc4(a_mat, ax, bdeg, d_col):
    return pl.pallas_call(
        _tc4_body,
        grid=(_GN,),
        in_specs=[
            pl.BlockSpec((_NB, M), lambda i: (i, 0)),
            pl.BlockSpec((M, C), lambda i: (0, 0)),
            pl.BlockSpec((1, M), lambda i: (0, 0)),
            pl.BlockSpec((_NB, 1), lambda i: (i, 0)),
        ],
        out_specs=pl.BlockSpec((_NB, C), lambda i: (i, 0)),
        out_shape=jax.ShapeDtypeStruct((N, C), jnp.float32),
    )(a_mat, ax, bdeg, d_col)


def kernel(x, hyperedge_index, weight, att):
    x2d = x[0]
    idx_n = hyperedge_index[0]
    idx_e = hyperedge_index[1]
    a1 = att[0, :, :C].reshape(1, C)
    a2 = att[0, :, C:].reshape(1, C)

    h_flat = _build_h(idx_n, idx_e)
    h2d = h_flat.reshape(N, M)

    xw, d_col, es, bdeg, du = _tc1(x2d, weight, h2d)
    loss, bv = _tc2(es, bdeg)
    a_mat, ax = _tc3(h2d, xw, es, a1, a2)
    out2 = _tc4(a_mat, ax, bdeg, d_col)

    out = out2[None, :, :]
    constrain = (jnp.abs((du[0, 0] - bv[0, 0]) / float(E * C))
                 + loss[0, 0] / float((M + 1) ** 2))
    return out, constrain


# speedup vs baseline: 2.1833x; 1.0436x over previous
import functools

import jax
import jax.numpy as jnp
from jax import lax
from jax.experimental import pallas as pl
from jax.experimental.pallas import tpu as pltpu
from jax.experimental.pallas import tpu_sc as plsc

N = 10000
E = 160000
M = 256
C = 128
NEG_SLOPE = 0.2


_HALF_ROWS = N // 2
_HALF_WORDS = _HALF_ROWS * M
_DUMMY = _HALF_WORDS
_EPC = E // 16
_NGRP = _EPC // 16
_NROW = 80
_ZW = 16000


def _build_h_body(idx_n_hbm, idx_e_hbm, zeros_hbm, h_hbm, idx_n_v, idx_e_v,
                  fidx, ones_v, h_half, dsem, zsem):
    cid = lax.axis_index("c")
    sid = lax.axis_index("s")
    n0 = cid * _HALF_ROWS

    base_e = sid * _EPC
    cp_n = pltpu.async_copy(idx_n_hbm.at[pl.ds(base_e, _EPC)],
                            idx_n_v.at[pl.ds(0, _EPC)], zsem)
    cp_e = pltpu.async_copy(idx_e_hbm.at[pl.ds(base_e, _EPC)],
                            idx_e_v.at[pl.ds(0, _EPC)], zsem)

    for j in range(8):
        ones_v[pl.ds(j * 16, 16)] = jnp.ones((16,), jnp.float32)

    zbase = sid * (_HALF_WORDS // 16)
    zcp = pltpu.async_copy(zeros_hbm.at[pl.ds(zbase, _HALF_WORDS // 16)],
                           h_half.at[pl.ds(zbase, _HALF_WORDS // 16)], dsem)

    cp_n.wait()
    cp_e.wait()
    for j in range(15):
        idx_n_v[pl.ds(_EPC + j * 16, 16)] = jnp.full((16,), -1, jnp.int32)

    lane = lax.iota(jnp.int32, 16)

    def _fx(r, _):
        for q in range(8):
            i = r * 8 + q
            n = idx_n_v[pl.ds(i * 16, 16)]
            m = idx_e_v[pl.ds(i * 16, 16)]
            in_half = (n >= n0) & (n < n0 + _HALF_ROWS)
            dummy = _DUMMY + ((i * 16 + lane) & 2047)
            f = jnp.where(in_half, (n - n0) * M + m, dummy)
            fidx[r, pl.ds(q * 16, 16)] = f
        return 0
    lax.fori_loop(0, _NROW, _fx, 0)

    zcp.wait()

    plsc.subcore_barrier()

    def _fire(j, _):
        pltpu.async_copy(ones_v, h_half.at[fidx.at[j]], dsem, add=True)
        return 0
    lax.fori_loop(0, _NROW, _fire, 0)

    def _drain(j, _):
        pltpu.make_async_copy(ones_v, h_half.at[fidx.at[j]], dsem).wait()
        return 0
    lax.fori_loop(0, _NROW, _drain, 0)

    plsc.subcore_barrier()

    out_base = cid * _HALF_WORDS + sid * (_HALF_WORDS // 16)
    pltpu.sync_copy(h_half.at[pl.ds(sid * (_HALF_WORDS // 16),
                                    _HALF_WORDS // 16)],
                    h_hbm.at[pl.ds(out_base, _HALF_WORDS // 16)])


def _build_h(idx_n, idx_e):
    mesh = plsc.VectorSubcoreMesh(core_axis_name="c", subcore_axis_name="s")
    zeros_hbm = jnp.zeros((_HALF_WORDS,), jnp.float32)
    return pl.kernel(
        _build_h_body,
        out_type=jax.ShapeDtypeStruct((N * M,), jnp.float32),
        mesh=mesh,
        scratch_types=[
            pltpu.VMEM((_EPC + 240,), jnp.int32),
            pltpu.VMEM((_EPC + 240,), jnp.int32),
            pltpu.VMEM((_NROW, 128), jnp.int32),
            pltpu.VMEM((128,), jnp.float32),
            pltpu.VMEM_SHARED((_HALF_WORDS + 2048,), jnp.float32),
            pltpu.SemaphoreType.DMA,
            pltpu.SemaphoreType.DMA,
        ],
    )(idx_n, idx_e, zeros_hbm)


_NB = 2000
_GN = N // _NB
_KB = 32
_GK = M // _KB


def _tc1_body(x_ref, w_ref, h_ref, xw_ref, d_ref, es_ref, bdeg_ref, du_ref):
    i = pl.program_id(0)
    xw = jnp.dot(x_ref[...], w_ref[...], preferred_element_type=jnp.float32)
    xw_ref[...] = xw
    h = h_ref[...]
    dcol = jnp.sum(h, axis=1, keepdims=True)
    d_ref[...] = dcol
    es_p = lax.dot_general(h, xw, (((0,), (0,)), ((), ())),
                           preferred_element_type=jnp.float32,
                           precision=lax.Precision.HIGHEST)
    bdeg_p = jnp.sum(h, axis=0, keepdims=True)
    u = jnp.sum(xw, axis=1, keepdims=True)
    du_p = jnp.reshape(jnp.sum(dcol * u), (1, 1))

    @pl.when(i == 0)
    def _init():
        es_ref[...] = es_p
        bdeg_ref[...] = bdeg_p
        du_ref[...] = du_p

    @pl.when(i > 0)
    def _acc():
        es_ref[...] += es_p
        bdeg_ref[...] += bdeg_p
        du_ref[...] += du_p


def _tc1(x2d, weight, h2d):
    return pl.pallas_call(
        _tc1_body,
        grid=(_GN,),
        in_specs=[
            pl.BlockSpec((_NB, C), lambda i: (i, 0)),
            pl.BlockSpec((C, C), lambda i: (0, 0)),
            pl.BlockSpec((_NB, M), lambda i: (i, 0)),
        ],
        out_specs=[
            pl.BlockSpec((_NB, C), lambda i: (i, 0)),
            pl.BlockSpec((_NB, 1), lambda i: (i, 0)),
            pl.BlockSpec((M, C), lambda i: (0, 0)),
            pl.BlockSpec((1, M), lambda i: (0, 0)),
            pl.BlockSpec((1, 1), lambda i: (0, 0)),
        ],
        out_shape=[
            jax.ShapeDtypeStruct((N, C), jnp.float32),
            jax.ShapeDtypeStruct((N, 1), jnp.float32),
            jax.ShapeDtypeStruct((M, C), jnp.float32),
            jax.ShapeDtypeStruct((1, M), jnp.float32),
            jax.ShapeDtypeStruct((1, 1), jnp.float32),
        ],
    )(x2d, weight, h2d)


def _tc3_body(h_ref, xw_ref, es_ref, a1_ref, a2_ref, a_out_ref, ax_ref):
    i = pl.program_id(0)
    h = h_ref[...]
    xw = xw_ref[...]
    s_col = jnp.sum(xw * a1_ref[...], axis=1, keepdims=True)
    t_row = jnp.sum(es_ref[...] * a2_ref[...], axis=1)[None, :]
    r = s_col + t_row
    r = jnp.where(r >= 0, r, NEG_SLOPE * r)
    mask = h > 0
    rm = jnp.where(mask, r, -jnp.inf)
    nmax = jnp.max(rm, axis=1, keepdims=True)
    nmax = jnp.where(nmax > -3e38, nmax, 0.0)
    ex = jnp.where(mask, jnp.exp(r - nmax), 0.0)
    nsum = jnp.sum(h * ex, axis=1, keepdims=True)
    a = h * ex / (nsum + 1e-16)
    a_out_ref[...] = a
    ax_p = lax.dot_general(a, xw, (((0,), (0,)), ((), ())),
                           preferred_element_type=jnp.float32,
                           precision=lax.Precision.HIGHEST)

    @pl.when(i == 0)
    def _init():
        ax_ref[...] = ax_p

    @pl.when(i > 0)
    def _acc():
        ax_ref[...] += ax_p


def _tc3(h2d, xw, es, a1, a2):
    return pl.pallas_call(
        _tc3_body,
        grid=(_GN,),
        in_specs=[
            pl.BlockSpec((_NB, M), lambda i: (i, 0)),
            pl.BlockSpec((_NB, C), lambda i: (i, 0)),
            pl.BlockSpec((M, C), lambda i: (0, 0)),
            pl.BlockSpec((1, C), lambda i: (0, 0)),
            pl.BlockSpec((1, C), lambda i: (0, 0)),
        ],
        out_specs=[
            pl.BlockSpec((_NB, M), lambda i: (i, 0)),
            pl.BlockSpec((M, C), lambda i: (0, 0)),
        ],
        out_shape=[
            jax.ShapeDtypeStruct((N, M), jnp.float32),
            jax.ShapeDtypeStruct((M, C), jnp.float32),
        ],
    )(h2d, xw, es, a1, a2)


def _tc4_body(a_ref, ax_ref, bdeg_ref, d_ref, es_ref, esk_ref, out_ref,
              loss_ref, bv_ref):
    i = pl.program_id(0)
    bdeg = bdeg_ref[...]
    bn_row = jnp.where(bdeg > 0, 1.0 / bdeg, 0.0)
    a_scaled = a_ref[...] * bn_row
    o = jnp.dot(a_scaled, ax_ref[...], preferred_element_type=jnp.float32,
                precision=lax.Precision.HIGHEST)
    out_ref[...] = d_ref[...] * o

    es = es_ref[...]
    sq_row = jnp.sum(es * es, axis=1)[None, :]
    nq_row = jnp.sqrt(sq_row)
    esk = esk_ref[...]
    inner = lax.dot_general(esk, es, (((1,), (1,)), ((), ())),
                            preferred_element_type=jnp.float32,
                            precision=lax.Precision.HIGHEST)
    sq_col = jnp.sum(esk * esk, axis=1, keepdims=True)
    ssq = jnp.maximum(sq_col + sq_row - 2.0 * inner, 0.0)
    dist = jnp.sqrt(ssq + 1e-12)
    alpha = inner / (jnp.sqrt(sq_col) * nq_row)
    item = alpha * dist + (1.0 - alpha) * jnp.maximum(4.2 - dist, 0.0)
    loss_p = jnp.reshape(jnp.sum(jnp.abs(item)), (1, 1))

    @pl.when(i == 0)
    def _init():
        loss_ref[...] = loss_p
        v = jnp.sum(es, axis=1)[None, :]
        bv_ref[...] = jnp.reshape(jnp.sum(bdeg * v), (1, 1))

    @pl.when((i > 0) & (i < 4))
    def _acc():
        loss_ref[...] += loss_p


def _tc4(a_mat, ax, bdeg, d_col, es):
    return pl.pallas_call(
        _tc4_body,
        grid=(_GN,),
        in_specs=[
            pl.BlockSpec((_NB, M), lambda i: (i, 0)),
            pl.BlockSpec((M, C), lambda i: (0, 0)),
            pl.BlockSpec((1, M), lambda i: (0, 0)),
            pl.BlockSpec((_NB, 1), lambda i: (i, 0)),
            pl.BlockSpec((M, C), lambda i: (0, 0)),
            pl.BlockSpec((64, C), lambda i: (jnp.minimum(i, 3), 0)),
        ],
        out_specs=[
            pl.BlockSpec((_NB, C), lambda i: (i, 0)),
            pl.BlockSpec((1, 1), lambda i: (0, 0)),
            pl.BlockSpec((1, 1), lambda i: (0, 0)),
        ],
        out_shape=[
            jax.ShapeDtypeStruct((N, C), jnp.float32),
            jax.ShapeDtypeStruct((1, 1), jnp.float32),
            jax.ShapeDtypeStruct((1, 1), jnp.float32),
        ],
    )(a_mat, ax, bdeg, d_col, es, es)


def kernel(x, hyperedge_index, weight, att):
    x2d = x[0]
    idx_n = hyperedge_index[0]
    idx_e = hyperedge_index[1]
    a1 = att[0, :, :C].reshape(1, C)
    a2 = att[0, :, C:].reshape(1, C)

    h_flat = _build_h(idx_n, idx_e)
    h2d = h_flat.reshape(N, M)

    xw, d_col, es, bdeg, du = _tc1(x2d, weight, h2d)
    a_mat, ax = _tc3(h2d, xw, es, a1, a2)
    out2, loss, bv = _tc4(a_mat, ax, bdeg, d_col, es)

    out = out2[None, :, :]
    constrain = (jnp.abs((du[0, 0] - bv[0, 0]) / float(E * C))
                 + loss[0, 0] / float((M + 1) ** 2))
    return out, constrain
